# Initial kernel scaffold; baseline (speedup 1.0000x reference)
import jax
import jax.numpy as jnp
from jax.experimental import pallas as pl


def kernel(h, edge_index, W1, b1, W2, b2, gamma, beta):
    raise NotImplementedError("write your pallas kernel here")


# speedup vs baseline: 2.7850x; 2.7850x over previous
import functools

import jax
import jax.numpy as jnp
from jax import lax
from jax.experimental import pallas as pl
from jax.experimental.pallas import tpu as pltpu
from jax.experimental.pallas import tpu_sc as plsc

N_NODES = 10000
D = 128
L = 3
BN_EPS = 1e-5

NP = 10240
NC = 2
NS = 16
NW = NC * NS
CHUNK = 128
CHUNKS_PW = 80
EP = NW * CHUNKS_PW * CHUNK
ROWS_PT = NP // NS


def _sc_agg_body(x_hbm, src_hbm, dst_hbm, out_hbm, sidx, didx, rows, agg, sem):
    cid = lax.axis_index("c")
    sid = lax.axis_index("s")
    wid = cid * NS + sid

    zero16 = jnp.zeros((16,), jnp.float32)

    def zrow(i, carry):
        for k in range(D // 16):
            rows[i, pl.ds(k * 16, 16)] = zero16
        return carry

    lax.fori_loop(0, CHUNK, zrow, 0)
    for k in range(ROWS_PT // CHUNK):
        pltpu.sync_copy(rows, agg.at[pl.ds(sid * ROWS_PT + k * CHUNK, CHUNK)])
    plsc.subcore_barrier()

    pltpu.sync_copy(src_hbm.at[wid], sidx)
    pltpu.sync_copy(dst_hbm.at[wid], didx)

    def chunk_step(j, carry):
        pltpu.async_copy(x_hbm.at[sidx.at[j]], rows, sem).wait()
        pltpu.sync_copy(rows, agg.at[didx.at[j]], add=True)
        return carry

    lax.fori_loop(0, CHUNKS_PW, chunk_step, 0)
    plsc.subcore_barrier()

    for k in range(ROWS_PT // CHUNK):
        off = sid * ROWS_PT + k * CHUNK
        pltpu.sync_copy(agg.at[pl.ds(off, CHUNK)], rows)
        pltpu.sync_copy(rows, out_hbm.at[cid, pl.ds(off, CHUNK)])


_sc_agg = pl.kernel(
    _sc_agg_body,
    out_type=jax.ShapeDtypeStruct((NC, NP, D), jnp.float32),
    mesh=plsc.VectorSubcoreMesh(core_axis_name="c", subcore_axis_name="s"),
    scratch_types=[
        pltpu.VMEM((CHUNKS_PW, CHUNK), jnp.int32),
        pltpu.VMEM((CHUNKS_PW, CHUNK), jnp.int32),
        pltpu.VMEM((CHUNK, D), jnp.float32),
        pltpu.VMEM_SHARED((NP, D), jnp.float32),
        pltpu.SemaphoreType.DMA,
    ],
)


def _tc_layer_body(x_ref, p_ref, w1_ref, b1_ref, w2_ref, b2_ref, g_ref,
                   bt_ref, hs_ref, xn_ref):
    z = x_ref[:N_NODES, :] + p_ref[0, :N_NODES, :] + p_ref[1, :N_NODES, :]
    h1 = jnp.dot(z, w1_ref[:, :], preferred_element_type=jnp.float32)
    h1 = jnp.maximum(h1 + b1_ref[:, :], 0.0)
    z2 = jnp.dot(h1, w2_ref[:, :], preferred_element_type=jnp.float32)
    z2 = z2 + b2_ref[:, :]
    m = jnp.mean(z2, axis=0, keepdims=True)
    c = z2 - m
    v = jnp.mean(c * c, axis=0, keepdims=True)
    y = jnp.maximum(c * lax.rsqrt(v + BN_EPS) * g_ref[:, :] + bt_ref[:, :], 0.0)
    hs_ref[:, :] = y
    xn_ref[:N_NODES, :] = y
    xn_ref[N_NODES:, :] = jnp.zeros((NP - N_NODES, D), jnp.float32)


_tc_layer = pl.pallas_call(
    _tc_layer_body,
    out_shape=(
        jax.ShapeDtypeStruct((N_NODES, D), jnp.float32),
        jax.ShapeDtypeStruct((NP, D), jnp.float32),
    ),
)


def kernel(h, edge_index, W1, b1, W2, b2, gamma, beta):
    src = edge_index[0]
    dst = edge_index[1]
    pad_e = EP - src.shape[0]
    pad_idx = jnp.full((pad_e,), N_NODES, jnp.int32)
    src_p = jnp.concatenate([src, pad_idx]).reshape(NW, CHUNKS_PW, CHUNK)
    dst_p = jnp.concatenate([dst, pad_idx]).reshape(NW, CHUNKS_PW, CHUNK)
    x = jnp.pad(h, ((0, NP - N_NODES), (0, 0)))
    hs = []
    for i in range(L):
        parts = _sc_agg(x, src_p, dst_p)
        y, x = _tc_layer(x, parts, W1[i], b1[i].reshape(1, D), W2[i],
                         b2[i].reshape(1, D), gamma[i].reshape(1, D),
                         beta[i].reshape(1, D))
        hs.append(y)
    return jnp.stack(hs)


# speedup vs baseline: 3.1541x; 1.1325x over previous
import functools

import jax
import jax.numpy as jnp
from jax import lax
from jax.experimental import pallas as pl
from jax.experimental.pallas import tpu as pltpu
from jax.experimental.pallas import tpu_sc as plsc

N_NODES = 10000
D = 128
L = 3
BN_EPS = 1e-5

NP = 10240
NC = 2
NS = 16
NW = NC * NS
CHUNK = 128
CHUNKS_PW = 80
HALF = CHUNKS_PW // 2
EP = NW * CHUNKS_PW * CHUNK
NA = 10112
ROWS_PT = NA // NS
_PIECES = [(k * CHUNK, CHUNK) for k in range(ROWS_PT // CHUNK)]
if ROWS_PT % CHUNK:
    _PIECES.append((ROWS_PT - ROWS_PT % CHUNK, ROWS_PT % CHUNK))


def _sc_agg_body(x_hbm, src_hbm, dst_hbm, out_hbm, sidx, didx, rows0, rows1,
                 agg, sem0, sem1):
    cid = lax.axis_index("c")
    sid = lax.axis_index("s")
    wid = cid * NS + sid

    zero16 = jnp.zeros((16,), jnp.float32)

    def zrow(i, carry):
        for k in range(D // 16):
            rows0[i, pl.ds(k * 16, 16)] = zero16
        return carry

    lax.fori_loop(0, CHUNK, zrow, 0)
    for off, n in _PIECES:
        pltpu.sync_copy(rows0.at[pl.ds(0, n)],
                        agg.at[pl.ds(sid * ROWS_PT + off, n)])
    plsc.subcore_barrier()

    def gather(j, buf, sem):
        return pltpu.make_async_copy(x_hbm.at[sidx.at[j]], buf, sem)

    for p in range(2):
        pltpu.sync_copy(src_hbm.at[wid, pl.ds(p * HALF, HALF)], sidx)
        pltpu.sync_copy(dst_hbm.at[wid, pl.ds(p * HALF, HALF)], didx)
        gather(0, rows0, sem0).start()

        def pair_step(t, carry):
            j0 = 2 * t
            gather(j0 + 1, rows1, sem1).start()
            gather(j0, rows0, sem0).wait()
            pltpu.sync_copy(rows0, agg.at[didx.at[j0]], add=True)

            @pl.when(t + 1 < HALF // 2)
            def _():
                gather(j0 + 2, rows0, sem0).start()

            gather(j0 + 1, rows1, sem1).wait()
            pltpu.sync_copy(rows1, agg.at[didx.at[j0 + 1]], add=True)
            return carry

        lax.fori_loop(0, HALF // 2, pair_step, 0)

    plsc.subcore_barrier()

    for off, n in _PIECES:
        pltpu.sync_copy(agg.at[pl.ds(sid * ROWS_PT + off, n)],
                        rows0.at[pl.ds(0, n)])
        pltpu.sync_copy(rows0.at[pl.ds(0, n)],
                        out_hbm.at[cid, pl.ds(sid * ROWS_PT + off, n)])


_sc_agg = pl.kernel(
    _sc_agg_body,
    out_type=jax.ShapeDtypeStruct((NC, NA, D), jnp.float32),
    mesh=plsc.VectorSubcoreMesh(core_axis_name="c", subcore_axis_name="s"),
    scratch_types=[
        pltpu.VMEM((HALF, CHUNK), jnp.int32),
        pltpu.VMEM((HALF, CHUNK), jnp.int32),
        pltpu.VMEM((CHUNK, D), jnp.float32),
        pltpu.VMEM((CHUNK, D), jnp.float32),
        pltpu.VMEM_SHARED((NA, D), jnp.float32),
        pltpu.SemaphoreType.DMA,
        pltpu.SemaphoreType.DMA,
    ],
)


def _tc_layer_body(x_ref, p_ref, w1_ref, b1_ref, w2_ref, b2_ref, g_ref,
                   bt_ref, hs_ref, xn_ref):
    z = x_ref[:N_NODES, :] + p_ref[0, :N_NODES, :] + p_ref[1, :N_NODES, :]
    h1 = jnp.dot(z, w1_ref[:, :], preferred_element_type=jnp.float32)
    h1 = jnp.maximum(h1 + b1_ref[:, :], 0.0)
    z2 = jnp.dot(h1, w2_ref[:, :], preferred_element_type=jnp.float32)
    z2 = z2 + b2_ref[:, :]
    m = jnp.mean(z2, axis=0, keepdims=True)
    c = z2 - m
    v = jnp.mean(c * c, axis=0, keepdims=True)
    y = jnp.maximum(c * lax.rsqrt(v + BN_EPS) * g_ref[:, :] + bt_ref[:, :], 0.0)
    hs_ref[:, :] = y
    xn_ref[:N_NODES, :] = y
    xn_ref[N_NODES:, :] = jnp.zeros((NP - N_NODES, D), jnp.float32)


_tc_layer = pl.pallas_call(
    _tc_layer_body,
    out_shape=(
        jax.ShapeDtypeStruct((N_NODES, D), jnp.float32),
        jax.ShapeDtypeStruct((NP, D), jnp.float32),
    ),
)


def kernel(h, edge_index, W1, b1, W2, b2, gamma, beta):
    src = edge_index[0]
    dst = edge_index[1]
    pad_e = EP - src.shape[0]
    src_p = jnp.concatenate([src, jnp.full((pad_e,), N_NODES, jnp.int32)])
    dst_p = jnp.concatenate([dst, jnp.full((pad_e,), N_NODES, jnp.int32)])
    src_p = src_p.reshape(NW, CHUNKS_PW, CHUNK)
    dst_p = dst_p.reshape(NW, CHUNKS_PW, CHUNK)
    x = jnp.pad(h, ((0, NP - N_NODES), (0, 0)))
    hs = []
    for i in range(L):
        parts = _sc_agg(x, src_p, dst_p)
        y, x = _tc_layer(x, parts, W1[i], b1[i].reshape(1, D), W2[i],
                         b2[i].reshape(1, D), gamma[i].reshape(1, D),
                         beta[i].reshape(1, D))
        hs.append(y)
    return jnp.stack(hs)


# speedup vs baseline: 8.0313x; 2.5463x over previous
import jax
import jax.numpy as jnp
from jax import lax
from jax.experimental import pallas as pl
from jax.experimental.pallas import tpu as pltpu
from jax.experimental.pallas import tpu_sc as plsc

N_NODES = 10000
D = 128
L = 3
BN_EPS = 1e-5

NP = 10240
NC = 2
NS = 16
DH = D // NC
CHUNK = 128
CHUNKS_PT = 160
PASS = 40
EP = NS * CHUNKS_PT * CHUNK
NA = 10112
ROWS_PT = NA // NS
XROWS_PT = NP // NS
_PIECES = [(k * CHUNK, CHUNK) for k in range(ROWS_PT // CHUNK)]
if ROWS_PT % CHUNK:
    _PIECES.append((ROWS_PT - ROWS_PT % CHUNK, ROWS_PT % CHUNK))


def _sc_agg_body(x_hbm, src_hbm, dst_hbm, out_hbm, sidx, didx, rows0, rows1,
                 rows2, rows3, xsp, agg, sg0, sg1, sg2, sg3):
    rows = [rows0, rows1, rows2, rows3]
    semg = [sg0, sg1, sg2, sg3]
    cid = lax.axis_index("c")
    sid = lax.axis_index("s")

    with jax.named_scope("stage_x"):
        for k in range(XROWS_PT // CHUNK):
            off = sid * XROWS_PT + k * CHUNK
            pltpu.sync_copy(
                x_hbm.at[pl.ds(off, CHUNK), pl.ds(cid * DH, DH)], rows1)
            pltpu.sync_copy(rows1, xsp.at[pl.ds(off, CHUNK)])

    zero16 = jnp.zeros((16,), jnp.float32)

    def zrow(i, carry):
        for k in range(DH // 16):
            rows0[i, pl.ds(k * 16, 16)] = zero16
        return carry

    with jax.named_scope("zero_agg"):
        lax.fori_loop(0, CHUNK, zrow, 0)
        for off, n in _PIECES:
            pltpu.sync_copy(rows0.at[pl.ds(0, n)],
                            agg.at[pl.ds(sid * ROWS_PT + off, n)])
    plsc.subcore_barrier()

    NBUF = 4

    def gather(j, b):
        return pltpu.make_async_copy(xsp.at[sidx.at[j]], rows[b], semg[b])

    for p in range(CHUNKS_PT // PASS):
        pltpu.sync_copy(src_hbm.at[sid, pl.ds(p * PASS, PASS)], sidx)
        pltpu.sync_copy(dst_hbm.at[sid, pl.ds(p * PASS, PASS)], didx)
        for b in range(NBUF):
            gather(b, b).start()

        def ring_step(t, carry):
            j0 = NBUF * t
            for b in range(NBUF):
                gather(j0 + b, b).wait()
                pltpu.sync_copy(rows[b], agg.at[didx.at[j0 + b]], add=True)

                @pl.when(j0 + b + NBUF < PASS)
                def _(b=b, j0=j0):
                    gather(j0 + b + NBUF, b).start()
            return carry

        lax.fori_loop(0, PASS // NBUF, ring_step, 0)

    plsc.subcore_barrier()

    with jax.named_scope("writeback"):
        for off, n in _PIECES:
            pltpu.sync_copy(agg.at[pl.ds(sid * ROWS_PT + off, n)],
                            rows0.at[pl.ds(0, n)])
            pltpu.sync_copy(
                rows0.at[pl.ds(0, n)],
                out_hbm.at[pl.ds(sid * ROWS_PT + off, n),
                           pl.ds(cid * DH, DH)])


_sc_agg = pl.kernel(
    _sc_agg_body,
    out_type=jax.ShapeDtypeStruct((NA, D), jnp.float32),
    mesh=plsc.VectorSubcoreMesh(core_axis_name="c", subcore_axis_name="s"),
    scratch_types=[
        pltpu.VMEM((PASS, CHUNK), jnp.int32),
        pltpu.VMEM((PASS, CHUNK), jnp.int32),
        pltpu.VMEM((CHUNK, DH), jnp.float32),
        pltpu.VMEM((CHUNK, DH), jnp.float32),
        pltpu.VMEM((CHUNK, DH), jnp.float32),
        pltpu.VMEM((CHUNK, DH), jnp.float32),
        pltpu.VMEM_SHARED((NP, DH), jnp.float32),
        pltpu.VMEM_SHARED((NA, DH), jnp.float32),
        pltpu.SemaphoreType.DMA,
        pltpu.SemaphoreType.DMA,
        pltpu.SemaphoreType.DMA,
        pltpu.SemaphoreType.DMA,
    ],
    compiler_params=pltpu.CompilerParams(use_tc_tiling_on_sc=False),
)


def _tc_layer_body(x_ref, p_ref, w1_ref, b1_ref, w2_ref, b2_ref, g_ref,
                   bt_ref, hs_ref, xn_ref):
    z = x_ref[:N_NODES, :] + p_ref[:N_NODES, :]
    h1 = jnp.dot(z, w1_ref[:, :], preferred_element_type=jnp.float32)
    h1 = jnp.maximum(h1 + b1_ref[:, :], 0.0)
    z2 = jnp.dot(h1, w2_ref[:, :], preferred_element_type=jnp.float32)
    z2 = z2 + b2_ref[:, :]
    m = jnp.mean(z2, axis=0, keepdims=True)
    c = z2 - m
    v = jnp.mean(c * c, axis=0, keepdims=True)
    y = jnp.maximum(c * lax.rsqrt(v + BN_EPS) * g_ref[:, :] + bt_ref[:, :], 0.0)
    hs_ref[:, :] = y
    xn_ref[:N_NODES, :] = y
    xn_ref[N_NODES:, :] = jnp.zeros((NP - N_NODES, D), jnp.float32)


_tc_layer = pl.pallas_call(
    _tc_layer_body,
    out_shape=(
        jax.ShapeDtypeStruct((N_NODES, D), jnp.float32),
        jax.ShapeDtypeStruct((NP, D), jnp.float32),
    ),
)


def kernel(h, edge_index, W1, b1, W2, b2, gamma, beta):
    src = edge_index[0]
    dst = edge_index[1]
    pad_e = EP - src.shape[0]
    pad_idx = jnp.full((pad_e,), N_NODES, jnp.int32)
    src_p = jnp.concatenate([src, pad_idx]).reshape(NS, CHUNKS_PT, CHUNK)
    dst_p = jnp.concatenate([dst, pad_idx]).reshape(NS, CHUNKS_PT, CHUNK)
    x = jnp.pad(h, ((0, NP - N_NODES), (0, 0)))
    hs = []
    for i in range(L):
        parts = _sc_agg(x, src_p, dst_p)
        y, x = _tc_layer(x, parts, W1[i], b1[i].reshape(1, D), W2[i],
                         b2[i].reshape(1, D), gamma[i].reshape(1, D),
                         beta[i].reshape(1, D))
        hs.append(y)
    return jnp.stack(hs)
